# baseline (speedup 1.0000x reference)
import jax
import jax.numpy as jnp
from jax.experimental import pallas as pl
from jax.experimental.pallas import tpu as pltpu

N = 4096
H = 1024
D = 256
TAU = 0.8
LAM = 0.5
EPS = 1e-8

BR = 256
BI = 128
BC = 1024
NC = N // BC
NI = N // BI
NB = N // BR
LANES = 128
LOG2E = 1.4426950408889634


def _lane_groups(x):
    acc = x[:, 0:LANES]
    for q in range(1, BC // LANES):
        acc = acc + x[:, q * LANES:(q + 1) * LANES]
    return acc


def _body(za_ref, zb_ref, W1_ref, b1_ref, W2_ref, b2_ref, pos_ref, neg_ref,
          loss_ref, out_ref, s_ref, acc_ref):
    t = pl.program_id(0)

    @pl.when(t == 0)
    def _init():
        acc_ref[0] = 0.0

    @pl.when(t < NB)
    def _proj():
        W1 = W1_ref[...]
        W2 = W2_ref[...]
        b1 = b1_ref[...]
        b2 = b2_ref[...]
        scale = (LOG2E / TAU) ** 0.5
        for idx, x_ref in ((0, za_ref), (1, zb_ref)):
            x = x_ref[...]
            h = jnp.dot(x, W1, preferred_element_type=jnp.float32) + b1
            h = jnp.where(h > 0, h, jnp.exp(h) - 1.0)
            p = jnp.dot(h, W2, preferred_element_type=jnp.float32) + b2
            nrm = jnp.sqrt(jnp.sum(p * p, axis=1, keepdims=True))
            out_ref[:, idx * D:(idx + 1) * D] = p
            s_ref[pl.ds(t * BR, BR), idx * D:(idx + 1) * D] = (
                p * (scale / nrm)).astype(jnp.bfloat16)

    @pl.when(t >= NB)
    def _sim():
        i = t - NB
        zi = s_ref[pl.ds(i * BI, BI), :]
        za_i = zi[:, 0:D]
        zb_i = zi[:, D:2 * D]
        accs = [jnp.zeros((BI, LANES), jnp.float32) for _ in range(6)]
        for c in range(NC):
            chunk = s_ref[pl.ds(c * BC, BC), :]
            za_c = chunk[:, 0:D]
            zb_c = chunk[:, D:2 * D]
            pos_c = pos_ref[:, c * BC:(c + 1) * BC]
            neg_c = neg_ref[:, c * BC:(c + 1) * BC]
            E1 = jnp.exp2(jax.lax.dot_general(
                za_i, zb_c, (((1,), (1,)), ((), ())),
                preferred_element_type=jnp.float32))
            E2 = jnp.exp2(jax.lax.dot_general(
                zb_i, za_c, (((1,), (1,)), ((), ())),
                preferred_element_type=jnp.float32))
            accs[0] = accs[0] + _lane_groups(E1)
            accs[1] = accs[1] + _lane_groups(E1 * pos_c)
            accs[2] = accs[2] + _lane_groups(E1 * neg_c)
            accs[3] = accs[3] + _lane_groups(E2)
            accs[4] = accs[4] + _lane_groups(E2 * pos_c)
            accs[5] = accs[5] + _lane_groups(E2 * neg_c)
        ra, Pa, Qa, cb, Pb, Qb = [jnp.sum(a, axis=1) for a in accs]
        lra = jnp.log(ra + EPS)
        lcb = jnp.log(cb + EPS)
        pos_part = (jnp.sum(lra - jnp.log(Pa))
                    + jnp.sum(lcb - jnp.log(Pb)))
        neg_part = (jnp.sum(lra + jnp.log(ra / (ra + EPS) + EPS)
                            - jnp.log(Qa))
                    + jnp.sum(lcb + jnp.log(cb / (cb + EPS) + EPS)
                              - jnp.log(Qb)))
        acc_ref[0] += LAM * pos_part + (1.0 - LAM) * neg_part

        @pl.when(t == NB + NI - 1)
        def _fin():
            loss_ref[...] = jnp.full((1, 1), acc_ref[0] * (1.0 / N),
                                     jnp.float32)


@jax.jit
def kernel(za, zb, pos, neg, W1, b1, W2, b2):
    loss, out2 = pl.pallas_call(
        _body,
        grid=(NB + NI,),
        in_specs=[
            pl.BlockSpec((BR, H), lambda t: (jnp.minimum(t, NB - 1), 0)),
            pl.BlockSpec((BR, H), lambda t: (jnp.minimum(t, NB - 1), 0)),
            pl.BlockSpec((H, H), lambda t: (0, 0)),
            pl.BlockSpec((H,), lambda t: (0,)),
            pl.BlockSpec((H, D), lambda t: (0, 0)),
            pl.BlockSpec((D,), lambda t: (0,)),
            pl.BlockSpec((BI, N), lambda t: (jnp.maximum(t - NB, 0), 0)),
            pl.BlockSpec((BI, N), lambda t: (jnp.maximum(t - NB, 0), 0)),
        ],
        out_specs=[
            pl.BlockSpec((1, 1), lambda t: (0, 0)),
            pl.BlockSpec((BR, 2 * D), lambda t: (jnp.minimum(t, NB - 1), 0)),
        ],
        out_shape=[
            jax.ShapeDtypeStruct((1, 1), jnp.float32),
            jax.ShapeDtypeStruct((N, 2 * D), jnp.float32),
        ],
        scratch_shapes=[
            pltpu.VMEM((N, 2 * D), jnp.bfloat16),
            pltpu.SMEM((1,), jnp.float32),
        ],
        compiler_params=pltpu.CompilerParams(
            dimension_semantics=("arbitrary",),
        ),
    )(za, zb, W1, b1, W2, b2, pos, neg)

    return jnp.reshape(loss, ()), out2


# speedup vs baseline: 1.2105x; 1.2105x over previous
import jax
import jax.numpy as jnp
from jax.experimental import pallas as pl
from jax.experimental.pallas import tpu as pltpu

N = 4096
H = 1024
D = 256
TAU = 0.8
LAM = 0.5
EPS = 1e-8

BR = 512
BI = 256
BC = 1024
NC = N // BC
NI = N // BI
NB = N // BR
LANES = 128
LOG2E = 1.4426950408889634


def _lane_groups(x):
    acc = x[:, 0:LANES]
    for q in range(1, BC // LANES):
        acc = acc + x[:, q * LANES:(q + 1) * LANES]
    return acc


def _body(za_ref, zb_ref, W1_ref, b1_ref, W2_ref, b2_ref, pos_ref, neg_ref,
          loss_ref, out_ref, s_ref, acc_ref):
    t = pl.program_id(0)

    @pl.when(t == 0)
    def _init():
        acc_ref[0] = 0.0

    @pl.when(t < NB)
    def _proj():
        W1 = W1_ref[...]
        W2 = W2_ref[...]
        b1 = b1_ref[...]
        b2 = b2_ref[...]
        scale = (LOG2E / TAU) ** 0.5
        for idx, x_ref in ((0, za_ref), (1, zb_ref)):
            x = x_ref[...]
            h = jnp.dot(x, W1, preferred_element_type=jnp.float32) + b1
            h = jnp.where(h > 0, h, jnp.exp(h) - 1.0)
            p = jnp.dot(h, W2, preferred_element_type=jnp.float32) + b2
            nrm = jnp.sqrt(jnp.sum(p * p, axis=1, keepdims=True))
            out_ref[:, idx * D:(idx + 1) * D] = p
            s_ref[pl.ds(t * BR, BR), idx * D:(idx + 1) * D] = (
                p * (scale / nrm)).astype(jnp.bfloat16)

    @pl.when(t >= NB)
    def _sim():
        i = t - NB
        zi = s_ref[pl.ds(i * BI, BI), :]
        za_i = zi[:, 0:D]
        zb_i = zi[:, D:2 * D]
        accs = [jnp.zeros((BI, LANES), jnp.float32) for _ in range(6)]
        for c in range(NC):
            chunk = s_ref[pl.ds(c * BC, BC), :]
            za_c = chunk[:, 0:D]
            zb_c = chunk[:, D:2 * D]
            pos_c = pos_ref[:, c * BC:(c + 1) * BC]
            neg_c = neg_ref[:, c * BC:(c + 1) * BC]
            E1 = jnp.exp2(jax.lax.dot_general(
                za_i, zb_c, (((1,), (1,)), ((), ())),
                preferred_element_type=jnp.float32))
            E2 = jnp.exp2(jax.lax.dot_general(
                zb_i, za_c, (((1,), (1,)), ((), ())),
                preferred_element_type=jnp.float32))
            accs[0] = accs[0] + _lane_groups(E1)
            accs[1] = accs[1] + _lane_groups(E1 * pos_c)
            accs[2] = accs[2] + _lane_groups(E1 * neg_c)
            accs[3] = accs[3] + _lane_groups(E2)
            accs[4] = accs[4] + _lane_groups(E2 * pos_c)
            accs[5] = accs[5] + _lane_groups(E2 * neg_c)
        ra, Pa, Qa, cb, Pb, Qb = [jnp.sum(a, axis=1) for a in accs]
        lra = jnp.log(ra + EPS)
        lcb = jnp.log(cb + EPS)
        pos_part = (jnp.sum(lra - jnp.log(Pa))
                    + jnp.sum(lcb - jnp.log(Pb)))
        neg_part = (jnp.sum(lra + jnp.log(ra / (ra + EPS) + EPS)
                            - jnp.log(Qa))
                    + jnp.sum(lcb + jnp.log(cb / (cb + EPS) + EPS)
                              - jnp.log(Qb)))
        acc_ref[0] += LAM * pos_part + (1.0 - LAM) * neg_part

        @pl.when(t == NB + NI - 1)
        def _fin():
            loss_ref[...] = jnp.full((1, 1), acc_ref[0] * (1.0 / N),
                                     jnp.float32)


@jax.jit
def kernel(za, zb, pos, neg, W1, b1, W2, b2):
    loss, out2 = pl.pallas_call(
        _body,
        grid=(NB + NI,),
        in_specs=[
            pl.BlockSpec((BR, H), lambda t: (jnp.minimum(t, NB - 1), 0)),
            pl.BlockSpec((BR, H), lambda t: (jnp.minimum(t, NB - 1), 0)),
            pl.BlockSpec((H, H), lambda t: (0, 0)),
            pl.BlockSpec((H,), lambda t: (0,)),
            pl.BlockSpec((H, D), lambda t: (0, 0)),
            pl.BlockSpec((D,), lambda t: (0,)),
            pl.BlockSpec((BI, N), lambda t: (jnp.maximum(t - NB, 0), 0)),
            pl.BlockSpec((BI, N), lambda t: (jnp.maximum(t - NB, 0), 0)),
        ],
        out_specs=[
            pl.BlockSpec((1, 1), lambda t: (0, 0)),
            pl.BlockSpec((BR, 2 * D), lambda t: (jnp.minimum(t, NB - 1), 0)),
        ],
        out_shape=[
            jax.ShapeDtypeStruct((1, 1), jnp.float32),
            jax.ShapeDtypeStruct((N, 2 * D), jnp.float32),
        ],
        scratch_shapes=[
            pltpu.VMEM((N, 2 * D), jnp.bfloat16),
            pltpu.SMEM((1,), jnp.float32),
        ],
        compiler_params=pltpu.CompilerParams(
            dimension_semantics=("arbitrary",),
        ),
    )(za, zb, W1, b1, W2, b2, pos, neg)

    return jnp.reshape(loss, ()), out2
